# baseline (speedup 1.0000x reference)
import functools

import jax
import jax.numpy as jnp
from jax import lax
from jax.experimental import pallas as pl
from jax.experimental.pallas import tpu as pltpu
from jax.experimental.pallas import tpu_sc as plsc

B = 4096
T = 200
D = 64
BF = B * T
NC = 2
NS = 16
NW = NC * NS
PER_W = BF // NW
SUB = 128
K = 8
CH = K * SUB
NCH = PER_W // CH
ROWS_128 = BF // SUB
PER_W_128 = PER_W // SUB


def _body(tok_hbm, pidx_hbm, table_hbm, pos_hbm, out_hbm, idx_v, pidx_v,
          rows_v, sem):
    cid = lax.axis_index("c")
    sid = lax.axis_index("s")
    wid = sid * NC + cid

    @pl.loop(0, NCH)
    def _chunk(c):
        r0 = wid * PER_W_128 + c * K
        pltpu.sync_copy(tok_hbm.at[pl.ds(r0, K)], idx_v)
        pltpu.sync_copy(pidx_hbm.at[pl.ds(r0, K)], pidx_v)
        pos_dmas = [
            pltpu.async_copy(pos_hbm.at[pidx_v.at[j]], rows_v.at[j], sem)
            for j in range(K)
        ]
        for d in pos_dmas:
            d.wait()
        tab_dmas = [
            pltpu.async_copy(table_hbm.at[idx_v.at[j]], rows_v.at[j], sem,
                             add=True)
            for j in range(K)
        ]
        for d in tab_dmas:
            d.wait()
        pltpu.sync_copy(rows_v, out_hbm.at[pl.ds(r0, K)])


@functools.partial(
    pl.kernel,
    out_type=jax.ShapeDtypeStruct((ROWS_128, SUB, D), jnp.float32),
    mesh=plsc.VectorSubcoreMesh(core_axis_name="c", subcore_axis_name="s"),
    scratch_types=[
        pltpu.VMEM((K, SUB), jnp.int32),
        pltpu.VMEM((K, SUB), jnp.int32),
        pltpu.VMEM((K, SUB, D), jnp.float32),
        pltpu.SemaphoreType.DMA,
    ],
    compiler_params=pltpu.CompilerParams(use_tc_tiling_on_sc=False),
)
def _lookup(tok_hbm, pidx_hbm, table_hbm, pos_hbm, out_hbm, idx_v, pidx_v,
            rows_v, sem):
    _body(tok_hbm, pidx_hbm, table_hbm, pos_hbm, out_hbm, idx_v, pidx_v,
          rows_v, sem)


def kernel(tokens, tokens_embed, positional_embed):
    tok2d = tokens.astype(jnp.int32).reshape(ROWS_128, SUB)
    pidx2d = jnp.tile(jnp.arange(T, dtype=jnp.int32), B).reshape(ROWS_128, SUB)
    out = _lookup(tok2d, pidx2d, tokens_embed, positional_embed)
    return out.reshape(B, T, D)


# speedup vs baseline: 1.4088x; 1.4088x over previous
import functools

import jax
import jax.numpy as jnp
from jax import lax
from jax.experimental import pallas as pl
from jax.experimental.pallas import tpu as pltpu
from jax.experimental.pallas import tpu_sc as plsc

B = 4096
T = 200
D = 64
BF = B * T
NC = 2
NS = 16
NW = NC * NS
PER_W = BF // NW
SUB = 100
K = 4
CH = K * SUB
NCH = PER_W // CH
PW_SUB = PER_W // SUB
NBUF = 3


def _body(tok_hbm, pidx_hbm, table_hbm, pos_hbm, out_hbm,
          pidx_v, pos_sh, idx0, idx1, idx2, rows0, rows1, rows2, *sems):
    idx_v = [idx0, idx1, idx2]
    rows_v = [rows0, rows1, rows2]
    sem_ld = sems[0:3]
    sem_tab = sems[3:6]
    sem_pos = sems[6:9]
    sem_out = sems[9:12]

    cid = lax.axis_index("c")
    sid = lax.axis_index("s")
    wid = sid * NC + cid
    r0w = wid * PW_SUB
    base = wid * PER_W

    pltpu.sync_copy(pidx_hbm, pidx_v)

    @pl.when(sid == 0)
    def _fill_pos():
        pltpu.sync_copy(pos_hbm, pos_sh)

    plsc.subcore_barrier()

    def fire_ld(j, b):
        pltpu.make_async_copy(
            tok_hbm.at[pl.ds(r0w + j * K, K)], idx_v[b], sem_ld[b]).start()

    def drain_ld(b):
        pltpu.make_async_copy(
            tok_hbm.at[pl.ds(0, K)], idx_v[b], sem_ld[b]).wait()

    def fire_tab(b):
        for q in range(K):
            pltpu.async_copy(table_hbm.at[idx_v[b].at[q]],
                             rows_v[b].at[pl.ds(q * SUB, SUB)], sem_tab[b])

    def drain_tab(b):
        for q in range(K):
            pltpu.make_async_copy(
                table_hbm.at[pl.ds(0, SUB)],
                rows_v[b].at[pl.ds(q * SUB, SUB)], sem_tab[b]).wait()

    def fire_pos(b):
        for q in range(K):
            pltpu.async_copy(pos_sh.at[pidx_v.at[q]],
                             rows_v[b].at[pl.ds(q * SUB, SUB)], sem_pos[b],
                             add=True)

    def drain_pos(b):
        for q in range(K):
            pltpu.make_async_copy(
                table_hbm.at[pl.ds(0, SUB)],
                rows_v[b].at[pl.ds(q * SUB, SUB)], sem_pos[b]).wait()

    def fire_out(j, b):
        pltpu.make_async_copy(
            rows_v[b], out_hbm.at[pl.ds(base + j * CH, CH)], sem_out[b]
        ).start()

    def drain_out(b):
        pltpu.make_async_copy(
            rows_v[b], out_hbm.at[pl.ds(base, CH)], sem_out[b]).wait()

    n_macro = (NCH + 3 + 2) // 3

    @pl.loop(0, n_macro)
    def _macro(m):
        for s in range(3):
            i = m * 3 + s

            bA = s

            @pl.when(i < NCH)
            def _a():
                fire_ld(i, bA)

            jB = i - 1
            bB = (s - 1) % 3

            @pl.when(jnp.logical_and(jB >= 0, jB < NCH))
            def _b():
                drain_ld(bB)

                @pl.when(jB >= NBUF)
                def _reuse():
                    drain_out(bB)

                fire_tab(bB)

            jC = i - 2
            bC = (s - 2) % 3

            @pl.when(jnp.logical_and(jC >= 0, jC < NCH))
            def _c():
                drain_tab(bC)
                fire_pos(bC)

            jD = i - 3
            bD = s

            @pl.when(jnp.logical_and(jD >= 0, jD < NCH))
            def _d():
                drain_pos(bD)
                fire_out(jD, bD)

    for j in range(NCH - NBUF, NCH):
        drain_out(j % 3)


@functools.partial(
    pl.kernel,
    out_type=jax.ShapeDtypeStruct((BF, D), jnp.float32),
    mesh=plsc.VectorSubcoreMesh(core_axis_name="c", subcore_axis_name="s"),
    scratch_types=[
        pltpu.VMEM((K, SUB), jnp.int32),
        pltpu.VMEM_SHARED((T, D), jnp.float32),
        pltpu.VMEM((K, SUB), jnp.int32),
        pltpu.VMEM((K, SUB), jnp.int32),
        pltpu.VMEM((K, SUB), jnp.int32),
        pltpu.VMEM((CH, D), jnp.float32),
        pltpu.VMEM((CH, D), jnp.float32),
        pltpu.VMEM((CH, D), jnp.float32),
    ] + [pltpu.SemaphoreType.DMA] * 12,
    compiler_params=pltpu.CompilerParams(use_tc_tiling_on_sc=False),
)
def _lookup(tok_hbm, pidx_hbm, table_hbm, pos_hbm, out_hbm, *scratch):
    _body(tok_hbm, pidx_hbm, table_hbm, pos_hbm, out_hbm, *scratch)


def kernel(tokens, tokens_embed, positional_embed):
    tok2d = tokens.astype(jnp.int32).reshape(BF // SUB, SUB)
    pidx = (jnp.arange(CH, dtype=jnp.int32) % T).reshape(K, SUB)
    out = _lookup(tok2d, pidx, tokens_embed, positional_embed)
    return out.reshape(B, T, D)


# speedup vs baseline: 1.4142x; 1.0038x over previous
import functools

import jax
import jax.numpy as jnp
from jax import lax
from jax.experimental import pallas as pl
from jax.experimental.pallas import tpu as pltpu
from jax.experimental.pallas import tpu_sc as plsc

B = 4096
T = 200
D = 64
BF = B * T
NC = 2
NS = 16
NW = NC * NS
PER_W = BF // NW
SUB = 40
K = 10
CH = K * SUB
NCH = PER_W // CH
TPW = B // NW
NBUF = 3
L = 16


def _body(tok_hbm, table_hbm, pos_hbm, out_hbm,
          pidx_v, pos_sh, idx0, idx1, idx2, rows0, rows1, rows2, *sems):
    idx_v = [idx0, idx1, idx2]
    rows_v = [rows0, rows1, rows2]
    sem_ld = sems[0:3]
    sem_tab = sems[3:6]
    sem_pos = sems[6:9]
    sem_out = sems[9:12]

    cid = lax.axis_index("c")
    sid = lax.axis_index("s")
    wid = sid * NC + cid
    row0 = wid * TPW

    for i in range(CH // L):
        v = lax.iota(jnp.int32, L) + (i * L)
        pidx_v[pl.ds(i * L, L)] = jnp.where(v >= T, v - T, v)

    @pl.when(sid == 0)
    def _fill_pos():
        pltpu.sync_copy(pos_hbm, pos_sh)

    plsc.subcore_barrier()

    def fire_ld(j, b):
        pltpu.make_async_copy(
            tok_hbm.at[pl.ds(row0 + 2 * j, 2)], idx_v[b], sem_ld[b]).start()

    def drain_ld(b):
        pltpu.make_async_copy(
            tok_hbm.at[pl.ds(0, 2)], idx_v[b], sem_ld[b]).wait()

    def fire_tab(b):
        for q in range(2):
            for h in range(5):
                pltpu.async_copy(
                    table_hbm.at[idx_v[b].at[q, pl.ds(h * SUB, SUB)]],
                    rows_v[b].at[q, pl.ds(h * SUB, SUB)], sem_tab[b])

    def drain_tab(b):
        for q in range(2):
            for h in range(5):
                pltpu.make_async_copy(
                    table_hbm.at[pl.ds(0, SUB)],
                    rows_v[b].at[q, pl.ds(h * SUB, SUB)], sem_tab[b]).wait()

    def fire_pos(b):
        for q in range(2):
            for h in range(5):
                pltpu.async_copy(
                    pos_sh.at[pidx_v.at[pl.ds((5 * q + h) * SUB, SUB)]],
                    rows_v[b].at[q, pl.ds(h * SUB, SUB)], sem_pos[b],
                    add=True)

    def drain_pos(b):
        for q in range(2):
            for h in range(5):
                pltpu.make_async_copy(
                    table_hbm.at[pl.ds(0, SUB)],
                    rows_v[b].at[q, pl.ds(h * SUB, SUB)], sem_pos[b]).wait()

    def fire_out(j, b):
        pltpu.make_async_copy(
            rows_v[b], out_hbm.at[pl.ds(row0 + 2 * j, 2)], sem_out[b]
        ).start()

    def drain_out(b):
        pltpu.make_async_copy(
            rows_v[b], out_hbm.at[pl.ds(row0, 2)], sem_out[b]).wait()

    n_macro = (NCH + 3 + 2) // 3

    @pl.loop(0, n_macro)
    def _macro(m):
        for s in range(3):
            i = m * 3 + s

            bA = s

            @pl.when(i < NCH)
            def _a():
                fire_ld(i, bA)

            jB = i - 1
            bB = (s - 1) % 3

            @pl.when(jnp.logical_and(jB >= 0, jB < NCH))
            def _b():
                drain_ld(bB)

                @pl.when(jB >= NBUF)
                def _reuse():
                    drain_out(bB)

                fire_tab(bB)

            jC = i - 2
            bC = (s - 2) % 3

            @pl.when(jnp.logical_and(jC >= 0, jC < NCH))
            def _c():
                drain_tab(bC)
                fire_pos(bC)

            jD = i - 3
            bD = s

            @pl.when(jnp.logical_and(jD >= 0, jD < NCH))
            def _d():
                drain_pos(bD)
                fire_out(jD, bD)

    for j in range(NCH - NBUF, NCH):
        drain_out(j % 3)


@functools.partial(
    pl.kernel,
    out_type=jax.ShapeDtypeStruct((B, T, D), jnp.float32),
    mesh=plsc.VectorSubcoreMesh(core_axis_name="c", subcore_axis_name="s"),
    scratch_types=[
        pltpu.VMEM((CH,), jnp.int32),
        pltpu.VMEM_SHARED((T, D), jnp.float32),
        pltpu.VMEM((2, T), jnp.int32),
        pltpu.VMEM((2, T), jnp.int32),
        pltpu.VMEM((2, T), jnp.int32),
        pltpu.VMEM((2, T, D), jnp.float32),
        pltpu.VMEM((2, T, D), jnp.float32),
        pltpu.VMEM((2, T, D), jnp.float32),
    ] + [pltpu.SemaphoreType.DMA] * 12,
    compiler_params=pltpu.CompilerParams(use_tc_tiling_on_sc=False),
)
def _lookup(tok_hbm, table_hbm, pos_hbm, out_hbm, *scratch):
    _body(tok_hbm, table_hbm, pos_hbm, out_hbm, *scratch)


def kernel(tokens, tokens_embed, positional_embed):
    return _lookup(tokens.astype(jnp.int32), tokens_embed, positional_embed)
